# baseline (speedup 1.0000x reference)
import functools

import jax
import jax.numpy as jnp
from jax import lax
from jax.experimental import pallas as pl
from jax.experimental.pallas import tpu as pltpu
from jax.experimental.pallas import tpu_sc as plsc

VOCAB = 1000000
D_MODEL = 64
D_PAD = 128
NUM_CORES = 2
NUM_SUBCORES = 16
NUM_WORKERS = NUM_CORES * NUM_SUBCORES
LANES = 16

UNROLL_D = 4
STRIPE = 128
NSTRIPES = VOCAB // STRIPE
TAIL = VOCAB - NSTRIPES * STRIPE

CHUNK = 512
NBUF = 2


def _make_transpose():
    mesh = plsc.VectorSubcoreMesh(core_axis_name="c", subcore_axis_name="s")
    n_iters = (NSTRIPES + NUM_WORKERS - 1) // NUM_WORKERS

    @functools.partial(
        pl.kernel,
        mesh=mesh,
        out_type=jax.ShapeDtypeStruct((VOCAB * D_MODEL,), jnp.float32),
        compiler_params=pltpu.CompilerParams(
            use_tc_tiling_on_sc=True, needs_layout_passes=False
        ),
        scratch_types=[
            pltpu.VMEM((D_MODEL, STRIPE), jnp.float32),
            pltpu.VMEM((D_MODEL, STRIPE), jnp.float32),
            pltpu.VMEM((D_MODEL * STRIPE,), jnp.float32),
            pltpu.VMEM((D_MODEL * STRIPE,), jnp.float32),
            pltpu.VMEM((TAIL * D_MODEL,), jnp.float32),
            pltpu.SemaphoreType.DMA,
            pltpu.SemaphoreType.DMA,
            pltpu.SemaphoreType.DMA,
            pltpu.SemaphoreType.DMA,
        ],
    )
    def transpose(wt_hbm, tail_hbm, out_hbm, src0, src1, dst0, dst1,
                  tailv, rs0, rs1, ws0, ws1):
        wid = lax.axis_index("s") * NUM_CORES + lax.axis_index("c")
        srcs = (src0, src1)
        dsts = (dst0, dst1)
        rsems = (rs0, rs1)
        wsems = (ws0, ws1)

        @pl.when(wid == 0)
        def _():
            pltpu.sync_copy(tail_hbm, tailv)
            pltpu.sync_copy(
                tailv, out_hbm.at[pl.ds(NSTRIPES * STRIPE * D_MODEL,
                                        TAIL * D_MODEL)]
            )

        iota64 = lax.iota(jnp.int32, LANES) * D_MODEL

        def stripe_of(i):
            return wid + i * NUM_WORKERS

        for b in range(NBUF):
            @pl.when(stripe_of(b) < NSTRIPES)
            def _():
                pltpu.async_copy(
                    wt_hbm.at[:, pl.ds(stripe_of(b) * STRIPE, STRIPE)],
                    srcs[b], rsems[b],
                )

        def body(i, carry):
            for b in range(NBUF):
                it = i * NBUF + b
                c = stripe_of(it)

                @pl.when(c < NSTRIPES)
                def _():
                    pltpu.make_async_copy(
                        wt_hbm.at[:, pl.ds(c * STRIPE, STRIPE)],
                        srcs[b], rsems[b],
                    ).wait()
                    @pl.when(it >= NBUF)
                    def _():
                        pltpu.make_async_copy(
                            dsts[b],
                            out_hbm.at[pl.ds(0, D_MODEL * STRIPE)],
                            wsems[b],
                        ).wait()

                    dviews = [
                        dsts[b].at[pl.ds(g * LANES * D_MODEL, LANES * D_MODEL)]
                        for g in range(STRIPE // LANES)
                    ]

                    @plsc.parallel_loop(0, D_MODEL, unroll=UNROLL_D)
                    def _(d):
                        idxd = iota64 + d
                        vecs = [
                            srcs[b][d, pl.ds(g * LANES, LANES)]
                            for g in range(STRIPE // LANES)
                        ]
                        for g in range(STRIPE // LANES):
                            plsc.store_scatter(dviews[g], [idxd], vecs[g])
                    pltpu.async_copy(
                        dsts[b],
                        out_hbm.at[pl.ds(c * STRIPE * D_MODEL,
                                         STRIPE * D_MODEL)],
                        wsems[b],
                    )
                    nxt = stripe_of(it + NBUF)

                    @pl.when(nxt < NSTRIPES)
                    def _():
                        pltpu.async_copy(
                            wt_hbm.at[:, pl.ds(nxt * STRIPE, STRIPE)],
                            srcs[b], rsems[b],
                        )

            return carry

        lax.fori_loop(0, (n_iters + NBUF - 1) // NBUF, body, 0)

        for b in range(NBUF):
            @pl.when(stripe_of(b) < NSTRIPES)
            def _():
                pltpu.make_async_copy(
                    dsts[b], out_hbm.at[pl.ds(0, D_MODEL * STRIPE)], wsems[b]
                ).wait()

    return transpose


@functools.lru_cache(maxsize=None)
def _make_lookup(B: int):
    assert B % (NUM_WORKERS * CHUNK) == 0
    b_per_w = B // NUM_WORKERS
    nchunks = b_per_w // CHUNK
    assert nchunks % NBUF == 0
    mesh = plsc.VectorSubcoreMesh(core_axis_name="c", subcore_axis_name="s")

    @functools.partial(
        pl.kernel,
        mesh=mesh,
        out_type=jax.ShapeDtypeStruct((B, D_PAD), jnp.float32),
        compiler_params=pltpu.CompilerParams(use_tc_tiling_on_sc=False),
        scratch_types=[
            pltpu.VMEM((b_per_w,), jnp.int32),
            pltpu.VMEM((CHUNK, D_MODEL), jnp.float32),
            pltpu.VMEM((CHUNK, D_MODEL), jnp.float32),
            pltpu.SemaphoreType.DMA,
            pltpu.SemaphoreType.DMA,
        ],
    )
    def lookup(idx_hbm, table_hbm, out_hbm, idx_v, rows0, rows1, sem0, sem1):
        wid = lax.axis_index("s") * NUM_CORES + lax.axis_index("c")
        base = wid * b_per_w
        rows = (rows0, rows1)
        sems = (sem0, sem1)

        pltpu.sync_copy(idx_hbm.at[wid], idx_v)

        for b in range(NBUF):
            pltpu.async_copy(
                table_hbm.at[idx_v.at[pl.ds(b * CHUNK, CHUNK)]],
                rows[b], sems[b],
            )

        def body(i, carry):
            j0 = i * NBUF
            for b in range(NBUF):
                j = j0 + b
                pltpu.make_async_copy(
                    table_hbm.at[idx_v.at[pl.ds(j * CHUNK, CHUNK)]],
                    rows[b], sems[b],
                ).wait()
                pltpu.sync_copy(
                    rows[b],
                    out_hbm.at[pl.ds(base + j * CHUNK, CHUNK),
                               pl.ds(0, D_MODEL)],
                )
                nxt = j + NBUF

                @pl.when(nxt < nchunks)
                def _():
                    pltpu.async_copy(
                        table_hbm.at[idx_v.at[pl.ds(nxt * CHUNK, CHUNK)]],
                        rows[b], sems[b],
                    )

            return carry

        lax.fori_loop(0, nchunks // NBUF, body, 0)

    return lookup


def kernel(token_ids, weight):
    s0, s1 = token_ids.shape
    B = s0 * s1
    idx = token_ids.astype(jnp.int32).reshape(NUM_WORKERS, B // NUM_WORKERS)
    tail = weight[NSTRIPES * STRIPE:].reshape(-1)
    table_flat = _make_transpose()(weight.T, tail)
    table = table_flat.reshape(VOCAB, D_MODEL)
    out = _make_lookup(B)(idx, table)
    return out[:, :D_MODEL].reshape(s0, s1, D_MODEL)


# speedup vs baseline: 1.5757x; 1.5757x over previous
import functools

import jax
import jax.numpy as jnp
from jax import lax
from jax.experimental import pallas as pl
from jax.experimental.pallas import tpu as pltpu
from jax.experimental.pallas import tpu_sc as plsc

D_MODEL = 64
D_PAD = 128
NUM_CORES = 2
NUM_SUBCORES = 16
NUM_WORKERS = NUM_CORES * NUM_SUBCORES
CHUNK = 512
NBUF = 2


@functools.lru_cache(maxsize=None)
def _make_lookup(B: int):
    assert B % (NUM_WORKERS * CHUNK) == 0
    b_per_w = B // NUM_WORKERS
    nchunks = b_per_w // CHUNK
    assert nchunks % NBUF == 0
    mesh = plsc.VectorSubcoreMesh(core_axis_name="c", subcore_axis_name="s")

    @functools.partial(
        pl.kernel,
        mesh=mesh,
        out_type=jax.ShapeDtypeStruct((B, D_PAD), jnp.float32),
        compiler_params=pltpu.CompilerParams(use_tc_tiling_on_sc=False),
        scratch_types=[
            pltpu.VMEM((b_per_w,), jnp.int32),
            pltpu.VMEM((CHUNK, D_MODEL), jnp.float32),
            pltpu.VMEM((CHUNK, D_MODEL), jnp.float32),
            pltpu.SemaphoreType.DMA,
            pltpu.SemaphoreType.DMA,
        ],
    )
    def lookup(idx_hbm, table_hbm, out_hbm, idx_v, rows0, rows1, sem0, sem1):
        wid = lax.axis_index("s") * NUM_CORES + lax.axis_index("c")
        base = wid * b_per_w
        rows = (rows0, rows1)
        sems = (sem0, sem1)

        pltpu.sync_copy(idx_hbm.at[wid], idx_v)

        for b in range(NBUF):
            pltpu.async_copy(
                table_hbm.at[idx_v.at[pl.ds(b * CHUNK, CHUNK)]],
                rows[b], sems[b],
            )

        def body(i, carry):
            j0 = i * NBUF
            for b in range(NBUF):
                j = j0 + b
                pltpu.make_async_copy(
                    table_hbm.at[idx_v.at[pl.ds(j * CHUNK, CHUNK)]],
                    rows[b], sems[b],
                ).wait()
                pltpu.sync_copy(
                    rows[b],
                    out_hbm.at[pl.ds(base + j * CHUNK, CHUNK),
                               pl.ds(0, D_MODEL)],
                )
                nxt = j + NBUF

                @pl.when(nxt < nchunks)
                def _():
                    pltpu.async_copy(
                        table_hbm.at[idx_v.at[pl.ds(nxt * CHUNK, CHUNK)]],
                        rows[b], sems[b],
                    )

            return carry

        lax.fori_loop(0, nchunks // NBUF, body, 0)

    return lookup


def kernel(token_ids, weight):
    s0, s1 = token_ids.shape
    B = s0 * s1
    idx = token_ids.astype(jnp.int32).reshape(NUM_WORKERS, B // NUM_WORKERS)
    out = _make_lookup(B)(idx, weight)
    return out[:, :D_MODEL].reshape(s0, s1, D_MODEL)
